# baseline (speedup 1.0000x reference)
import math

import jax
import jax.numpy as jnp
from jax.experimental import pallas as pl
from jax.experimental.pallas import tpu as pltpu

_K_CHUNKS = 31
_H1 = 512
_H2 = 256
_N_CHUNKS = 77

_P1_TI = 1
_P1_STEPS = 16
_P3_TI = 8

_VMEM_LIMIT = 100 * 1024 * 1024


def _gelu(x):
    return 0.5 * x * (1.0 + jax.lax.erf(x * (1.0 / math.sqrt(2.0))))


def _proj1_kernel(x_ref, w_ref, o_ref, acc_ref):
    kh = pl.program_id(0)
    k = pl.program_id(1)
    nk = pl.num_programs(1)

    @pl.when(k == 0)
    def _():
        acc_ref[...] = jnp.zeros_like(acc_ref)

    w_all = w_ref[...]
    acc = acc_ref[...]
    base = (kh * nk + k) * _P1_TI
    for i in range(_P1_TI):
        valid = base + i < _K_CHUNKS
        xi = jnp.where(valid, x_ref[i], 0.0).astype(jnp.bfloat16)
        wi = jnp.where(valid, w_all[1024 * i:1024 * (i + 1), :], 0.0)
        acc += jnp.dot(xi, wi.astype(jnp.bfloat16),
                       preferred_element_type=jnp.float32)
    acc_ref[...] = acc

    @pl.when(k == nk - 1)
    def _():
        o_ref[0] = acc


def _proj1(xt, w1):
    M = xt.shape[1]
    return pl.pallas_call(
        _proj1_kernel,
        out_shape=jax.ShapeDtypeStruct((2, M, _H1), jnp.float32),
        grid_spec=pltpu.PrefetchScalarGridSpec(
            num_scalar_prefetch=0,
            grid=(2, _P1_STEPS),
            in_specs=[
                pl.BlockSpec((_P1_TI, M, 1024),
                             lambda kh, k: (kh * _P1_STEPS + k, 0, 0)),
                pl.BlockSpec((_P1_TI * 1024, _H1),
                             lambda kh, k: (kh * _P1_STEPS + k, 0)),
            ],
            out_specs=pl.BlockSpec((1, M, _H1), lambda kh, k: (kh, 0, 0)),
            scratch_shapes=[pltpu.VMEM((M, _H1), jnp.float32)],
        ),
        compiler_params=pltpu.CompilerParams(
            dimension_semantics=("parallel", "arbitrary"),
            vmem_limit_bytes=_VMEM_LIMIT,
        ),
    )(xt, w1)


def _proj23_kernel(h1p_ref, b1_ref, w2_ref, b2_ref, w3_ref, b3_ref, o_ref):
    h1 = _gelu(h1p_ref[0] + h1p_ref[1] + b1_ref[...]).astype(jnp.bfloat16)
    h2 = _gelu(
        jnp.dot(h1, w2_ref[...].astype(jnp.bfloat16),
                preferred_element_type=jnp.float32)
        + b2_ref[...]
    ).astype(jnp.bfloat16)
    w3_all = w3_ref[...]
    b3_all = b3_ref[...]
    for i in range(_P3_TI):
        wv = w3_all[:, 1024 * i:1024 * (i + 1)].astype(jnp.bfloat16)
        val = jnp.dot(h2, wv, preferred_element_type=jnp.float32)
        o_ref[i] = val + b3_all[:, 1024 * i:1024 * (i + 1)]


def _proj23(h1p, b1, w2, b2, w3, b3):
    M = h1p.shape[1]
    nj = (_N_CHUNKS + _P3_TI - 1) // _P3_TI
    tn = _P3_TI * 1024
    return pl.pallas_call(
        _proj23_kernel,
        out_shape=jax.ShapeDtypeStruct((_N_CHUNKS, M, 1024), jnp.float32),
        grid_spec=pltpu.PrefetchScalarGridSpec(
            num_scalar_prefetch=0,
            grid=(nj,),
            in_specs=[
                pl.BlockSpec((2, M, _H1), lambda j: (0, 0, 0)),
                pl.BlockSpec((1, _H1), lambda j: (0, 0)),
                pl.BlockSpec((_H1, _H2), lambda j: (0, 0)),
                pl.BlockSpec((1, _H2), lambda j: (0, 0)),
                pl.BlockSpec((_H2, tn), lambda j: (0, j)),
                pl.BlockSpec((1, tn), lambda j: (0, j)),
            ],
            out_specs=pl.BlockSpec((_P3_TI, M, 1024), lambda j: (j, 0, 0)),
        ),
        compiler_params=pltpu.CompilerParams(
            dimension_semantics=("parallel",),
            vmem_limit_bytes=_VMEM_LIMIT,
        ),
    )(h1p, b1.reshape(1, _H1), w2, b2.reshape(1, _H2), w3,
      b3.reshape(1, _N_CHUNKS * 1024))


@jax.jit
def kernel(x, w1, b1, w2, b2, w3, b3):
    xt = jnp.transpose(x, (1, 0, 2))
    h1p = _proj1(xt, w1)
    out = _proj23(h1p, b1, w2, b2, w3, b3)
    return jnp.transpose(out, (1, 0, 2))


# speedup vs baseline: 1.1165x; 1.1165x over previous
import math

import jax
import jax.numpy as jnp
from jax.experimental import pallas as pl
from jax.experimental.pallas import tpu as pltpu

_K_CHUNKS = 31
_H1 = 512
_H2 = 256
_N_CHUNKS = 77

_P1_TI = 2
_P1_STEPS = 8
_P3_TI = 8

_VMEM_LIMIT = 100 * 1024 * 1024


def _gelu(x):
    return 0.5 * x * (1.0 + jax.lax.erf(x * (1.0 / math.sqrt(2.0))))


def _proj1_kernel(x_ref, w_ref, o_ref, acc_ref):
    kh = pl.program_id(0)
    k = pl.program_id(1)
    nk = pl.num_programs(1)

    @pl.when(k == 0)
    def _():
        acc_ref[...] = jnp.zeros_like(acc_ref)

    w_all = w_ref[...]
    acc = acc_ref[...]
    base = (kh * nk + k) * _P1_TI
    for i in range(_P1_TI):
        valid = base + i < _K_CHUNKS
        xi = jnp.where(valid, x_ref[i], 0.0).astype(jnp.bfloat16)
        wi = jnp.where(valid, w_all[1024 * i:1024 * (i + 1), :], 0.0)
        acc += jnp.dot(xi, wi.astype(jnp.bfloat16),
                       preferred_element_type=jnp.float32)
    acc_ref[...] = acc

    @pl.when(k == nk - 1)
    def _():
        o_ref[0] = acc


def _proj1(xt, w1):
    M = xt.shape[1]
    return pl.pallas_call(
        _proj1_kernel,
        out_shape=jax.ShapeDtypeStruct((2, M, _H1), jnp.float32),
        grid_spec=pltpu.PrefetchScalarGridSpec(
            num_scalar_prefetch=0,
            grid=(2, _P1_STEPS),
            in_specs=[
                pl.BlockSpec((_P1_TI, M, 1024),
                             lambda kh, k: (kh * _P1_STEPS + k, 0, 0)),
                pl.BlockSpec((_P1_TI * 1024, _H1),
                             lambda kh, k: (kh * _P1_STEPS + k, 0)),
            ],
            out_specs=pl.BlockSpec((1, M, _H1), lambda kh, k: (kh, 0, 0)),
            scratch_shapes=[pltpu.VMEM((M, _H1), jnp.float32)],
        ),
        compiler_params=pltpu.CompilerParams(
            dimension_semantics=("parallel", "arbitrary"),
            vmem_limit_bytes=_VMEM_LIMIT,
        ),
    )(xt, w1)


def _proj23_kernel(h1p_ref, b1_ref, w2_ref, b2_ref, w3_ref, b3_ref, o_ref):
    h1 = _gelu(h1p_ref[0] + h1p_ref[1] + b1_ref[...]).astype(jnp.bfloat16)
    h2 = _gelu(
        jnp.dot(h1, w2_ref[...].astype(jnp.bfloat16),
                preferred_element_type=jnp.float32)
        + b2_ref[...]
    ).astype(jnp.bfloat16)
    w3_all = w3_ref[...]
    b3_all = b3_ref[...]
    for i in range(_P3_TI):
        wv = w3_all[:, 1024 * i:1024 * (i + 1)].astype(jnp.bfloat16)
        val = jnp.dot(h2, wv, preferred_element_type=jnp.float32)
        o_ref[i] = val + b3_all[:, 1024 * i:1024 * (i + 1)]


def _proj23(h1p, b1, w2, b2, w3, b3):
    M = h1p.shape[1]
    nj = (_N_CHUNKS + _P3_TI - 1) // _P3_TI
    tn = _P3_TI * 1024
    return pl.pallas_call(
        _proj23_kernel,
        out_shape=jax.ShapeDtypeStruct((_N_CHUNKS, M, 1024), jnp.float32),
        grid_spec=pltpu.PrefetchScalarGridSpec(
            num_scalar_prefetch=0,
            grid=(nj,),
            in_specs=[
                pl.BlockSpec((2, M, _H1), lambda j: (0, 0, 0)),
                pl.BlockSpec((1, _H1), lambda j: (0, 0)),
                pl.BlockSpec((_H1, _H2), lambda j: (0, 0)),
                pl.BlockSpec((1, _H2), lambda j: (0, 0)),
                pl.BlockSpec((_H2, tn), lambda j: (0, j)),
                pl.BlockSpec((1, tn), lambda j: (0, j)),
            ],
            out_specs=pl.BlockSpec((_P3_TI, M, 1024), lambda j: (j, 0, 0)),
        ),
        compiler_params=pltpu.CompilerParams(
            dimension_semantics=("parallel",),
            vmem_limit_bytes=_VMEM_LIMIT,
        ),
    )(h1p, b1.reshape(1, _H1), w2, b2.reshape(1, _H2), w3,
      b3.reshape(1, _N_CHUNKS * 1024))


@jax.jit
def kernel(x, w1, b1, w2, b2, w3, b3):
    xt = jnp.transpose(x, (1, 0, 2))
    h1p = _proj1(xt, w1)
    out = _proj23(h1p, b1, w2, b2, w3, b3)
    return jnp.transpose(out, (1, 0, 2))
